# Initial kernel scaffold; baseline (speedup 1.0000x reference)
import jax
import jax.numpy as jnp
from jax.experimental import pallas as pl


def kernel(h, pos, batch, edge_index, W1a, b1a, W1b, b1b, W2a, b2a, W2b, b2b, Wc, bc):
    raise NotImplementedError("write your pallas kernel here")


# speedup vs baseline: 1.0026x; 1.0026x over previous
import jax
import jax.numpy as jnp
from jax.experimental import pallas as pl


def _final_matmul(pooled, Wc_pad, bc_pad):
    def body(x_ref, w_ref, b_ref, o_ref):
        o_ref[...] = jnp.dot(x_ref[...], w_ref[...],
                             preferred_element_type=jnp.float32) + b_ref[...]

    return pl.pallas_call(
        body,
        out_shape=jax.ShapeDtypeStruct((pooled.shape[0], Wc_pad.shape[1]), jnp.float32),
    )(pooled, Wc_pad, bc_pad)


def kernel(h, pos, batch, edge_index, W1a, b1a, W1b, b1b, W2a, b2a, W2b, b2b, Wc, bc):
    N = h.shape[0]
    G = 64
    src = edge_index[0]
    dst = edge_index[1]

    def pointnet_layer(feat, Wa, ba, Wb, bb):
        msg_in = jnp.concatenate([feat[src], pos[src] - pos[dst]], axis=-1)
        m = jnp.maximum(msg_in @ Wa + ba, 0.0) @ Wb + bb
        agg = jax.ops.segment_max(m, dst, num_segments=N)
        return jnp.where(jnp.isfinite(agg), agg, 0.0)

    h1 = jax.nn.relu(pointnet_layer(pos, W1a, b1a, W1b, b1b))
    h2 = jax.nn.relu(pointnet_layer(h1, W2a, b2a, W2b, b2b))
    pooled = jax.ops.segment_max(h2, batch, num_segments=G)
    pooled = jnp.where(jnp.isfinite(pooled), pooled, 0.0)

    Wc_pad = jnp.zeros((32, 128), jnp.float32).at[:, :3].set(Wc)
    bc_pad = jnp.zeros((1, 128), jnp.float32).at[0, :3].set(bc)
    out = _final_matmul(pooled, Wc_pad, bc_pad)
    return out[:, :3]


# speedup vs baseline: 2.6603x; 2.6533x over previous
import functools

import jax
import jax.numpy as jnp
from jax import lax
from jax.experimental import pallas as pl
from jax.experimental.pallas import tpu as pltpu
from jax.experimental.pallas import tpu_sc as plsc

NC = 2
NS = 16
NW = NC * NS
L = 16

BIN_SHIFT = 11
BIN_SIZE = 1 << BIN_SHIFT


def _mesh():
    return plsc.VectorSubcoreMesh(core_axis_name="c", subcore_axis_name="s")


def _wid():
    return lax.axis_index("s") * NC + lax.axis_index("c")


def _hist_kernel(E, NBINS):
    EPW = E // NW
    CH = 2000
    assert EPW % CH == 0 and CH % L == 0

    @functools.partial(
        pl.kernel, mesh=_mesh(),
        compiler_params=pltpu.CompilerParams(needs_layout_passes=False, use_tc_tiling_on_sc=False),
        out_type=jax.ShapeDtypeStruct((NW, NBINS * L), jnp.int32),
        scratch_types=[
            pltpu.VMEM((CH,), jnp.int32),
            pltpu.VMEM((NBINS * L,), jnp.int32),
        ],
    )
    def hist(dst_hbm, counts_hbm, dstbuf, cnt):
        w = _wid()
        lane = lax.iota(jnp.int32, L)
        zero = jnp.zeros((L,), jnp.int32)
        one = jnp.ones((L,), jnp.int32)
        for i in range(NBINS):
            cnt[pl.ds(i * L, L)] = zero
        base = w * EPW

        def chunk_body(c, _):
            pltpu.sync_copy(dst_hbm.at[pl.ds(base + c * CH, CH)], dstbuf)
            for g in range(CH // L):
                d = dstbuf[pl.ds(g * L, L)]
                b = lax.shift_right_logical(d, BIN_SHIFT)
                idxv = b * L + lane
                plsc.addupdate_scatter(cnt, [idxv], one)
            return 0

        lax.fori_loop(0, EPW // CH, chunk_body, 0)
        pltpu.sync_copy(cnt, counts_hbm.at[w])

    return hist


def _binscatter_kernel(E, E_pad, NBINS):
    EPW = E // NW
    CH = 2000
    assert EPW % CH == 0

    @functools.partial(
        pl.kernel, mesh=_mesh(),
        compiler_params=pltpu.CompilerParams(needs_layout_passes=False, use_tc_tiling_on_sc=False),
        out_type=[jax.ShapeDtypeStruct((E_pad,), jnp.int32),
                  jax.ShapeDtypeStruct((E_pad,), jnp.int32)],
        scratch_types=[
            pltpu.VMEM((CH,), jnp.int32),
            pltpu.VMEM((CH,), jnp.int32),
            pltpu.VMEM((CH,), jnp.int32),
            pltpu.VMEM((NBINS * L,), jnp.int32),
            pltpu.VMEM((NBINS * L,), jnp.int32),
            pltpu.SemaphoreType.DMA,
            pltpu.SemaphoreType.DMA,
        ],
    )
    def binscatter(src_hbm, dst_hbm, base_hbm, bsrc_hbm, bdst_hbm,
                   srcbuf, dstbuf, posbuf, run, basebuf, sem1, sem2):
        w = _wid()
        lane = lax.iota(jnp.int32, L)
        zero = jnp.zeros((L,), jnp.int32)
        one = jnp.ones((L,), jnp.int32)
        pltpu.sync_copy(base_hbm.at[w], basebuf)
        for i in range(NBINS):
            run[pl.ds(i * L, L)] = zero
        base = w * EPW

        def chunk_body(c, _):
            pltpu.sync_copy(src_hbm.at[pl.ds(base + c * CH, CH)], srcbuf)
            pltpu.sync_copy(dst_hbm.at[pl.ds(base + c * CH, CH)], dstbuf)
            for g in range(CH // L):
                d = dstbuf[pl.ds(g * L, L)]
                b = lax.shift_right_logical(d, BIN_SHIFT)
                idxv = b * L + lane
                bs = plsc.load_gather(basebuf, [idxv])
                cr = plsc.load_gather(run, [idxv])
                plsc.store_scatter(run, [idxv], cr + one)
                posbuf[pl.ds(g * L, L)] = bs + cr
            c1 = pltpu.async_copy(srcbuf, bsrc_hbm.at[posbuf], sem1)
            c2 = pltpu.async_copy(dstbuf, bdst_hbm.at[posbuf], sem2)
            c1.wait()
            c2.wait()
            return 0

        lax.fori_loop(0, EPW // CH, chunk_body, 0)

    return binscatter


def _gather_kernel(E, E_pad, N_pad):
    EPW = E // NW
    CH = 1000
    assert EPW % CH == 0

    @functools.partial(
        pl.kernel, mesh=_mesh(),
        compiler_params=pltpu.CompilerParams(needs_layout_passes=False, use_tc_tiling_on_sc=False),
        out_type=[jax.ShapeDtypeStruct((E_pad, 32), jnp.float32),
                  jax.ShapeDtypeStruct((E_pad, 32), jnp.float32)],
        scratch_types=[
            pltpu.VMEM((CH,), jnp.int32),
            pltpu.VMEM((CH,), jnp.int32),
            pltpu.VMEM((CH, 32), jnp.float32),
            pltpu.VMEM((CH, 32), jnp.float32),
            pltpu.SemaphoreType.DMA,
            pltpu.SemaphoreType.DMA,
        ],
    )
    def gather(u_hbm, v_hbm, bsrc_hbm, bdst_hbm, gu_hbm, gv_hbm,
               idxs, idxd, gubuf, gvbuf, sem1, sem2):
        w = _wid()
        base = w * EPW

        def chunk_body(c, _):
            off = base + c * CH
            pltpu.sync_copy(bsrc_hbm.at[pl.ds(off, CH)], idxs)
            pltpu.sync_copy(bdst_hbm.at[pl.ds(off, CH)], idxd)
            c1 = pltpu.async_copy(u_hbm.at[idxs], gubuf, sem1)
            c2 = pltpu.async_copy(v_hbm.at[idxd], gvbuf, sem2)
            c1.wait()
            c2.wait()
            pltpu.sync_copy(gubuf, gu_hbm.at[pl.ds(off, CH)])
            pltpu.sync_copy(gvbuf, gv_hbm.at[pl.ds(off, CH)])
            return 0

        lax.fori_loop(0, EPW // CH, chunk_body, 0)

    return gather


def _scatmax_kernel(E, E_pad, N_pad, NBINS, with_pool, G=64):
    CH = 512
    ROUNDS = (NBINS + NW - 1) // NW

    out_type = [jax.ShapeDtypeStruct((N_pad, 32), jnp.float32)]
    if with_pool:
        out_type.append(jax.ShapeDtypeStruct((NW * ROUNDS * G, 32), jnp.float32))
    scratch = [
        pltpu.VMEM((BIN_SIZE, 32), jnp.float32),
        pltpu.VMEM((CH // L, L), jnp.int32),
        pltpu.VMEM((CH, 32), jnp.float32),
        pltpu.VMEM((80,), jnp.int32),
    ]
    if with_pool:
        scratch.append(pltpu.VMEM((G, 32), jnp.float32))
        scratch.append(pltpu.VMEM((BIN_SIZE // L, L), jnp.int32))

    @functools.partial(pl.kernel, mesh=_mesh(), out_type=out_type,
                       compiler_params=pltpu.CompilerParams(needs_layout_passes=False, use_tc_tiling_on_sc=False),
                       scratch_types=scratch)
    def scatmax(*refs):
        if with_pool:
            (m_hbm, bdst_hbm, bstart_hbm, batch_hbm, h_hbm, pool_hbm,
             acc, dstbuf, mbuf, bsbuf, pacc, batchbuf) = refs
        else:
            (m_hbm, bdst_hbm, bstart_hbm, h_hbm,
             acc, dstbuf, mbuf, bsbuf) = refs
        w = _wid()
        zero = jnp.zeros((L,), jnp.float32)
        pltpu.sync_copy(bstart_hbm, bsbuf)
        for r in range(ROUNDS):
            bin_id = w + NW * r
            live = bin_id < NBINS

            @pl.when(live)
            def _process():
                def zbody(i, _):
                    for k in range(4):
                        acc[i * 2 + (k // 2), pl.ds((k % 2) * L, L)] = zero
                    return 0
                lax.fori_loop(0, BIN_SIZE // 2, zbody, 0)

                ebounds = bsbuf[pl.ds(bin_id, 16)]
                e0 = ebounds[0]
                e1 = ebounds[1]
                abase = e0 - jnp.bitwise_and(e0, L - 1)
                nch = (e1 - abase + CH - 1) // CH
                nodebase = bin_id * BIN_SIZE

                def chunk_body(c, _):
                    cstart = abase + c * CH
                    pltpu.sync_copy(
                        bdst_hbm.at[pl.ds(lax.shift_right_logical(cstart, 4),
                                          CH // L)], dstbuf)
                    pltpu.sync_copy(m_hbm.at[pl.ds(cstart, CH)], mbuf)
                    lo = jnp.maximum(e0, cstart) - cstart
                    hi = jnp.minimum(e1, cstart + CH) - cstart
                    glo = lax.shift_right_logical(lo, 4)
                    ghi = lax.shift_right_logical(hi + L - 1, 4)

                    def group_body(g, _):
                        dv = dstbuf[g, pl.ds(0, L)] - nodebase
                        jb = g * L
                        for k in range(L):
                            j = jb + k

                            @pl.when(jnp.logical_and(j >= lo, j < hi))
                            def _rmw():
                                dl = dv[k]
                                a0 = acc[dl, pl.ds(0, L)]
                                a1 = acc[dl, pl.ds(L, L)]
                                m0 = mbuf[j, pl.ds(0, L)]
                                m1 = mbuf[j, pl.ds(L, L)]
                                acc[dl, pl.ds(0, L)] = jnp.maximum(a0, m0)
                                acc[dl, pl.ds(L, L)] = jnp.maximum(a1, m1)
                        return 0

                    lax.fori_loop(glo, ghi, group_body, 0)
                    return 0

                lax.fori_loop(0, nch, chunk_body, 0)
                pltpu.sync_copy(acc, h_hbm.at[pl.ds(nodebase, BIN_SIZE)])

                if with_pool:
                    def pzbody(i, _):
                        pacc[i, pl.ds(0, L)] = zero
                        pacc[i, pl.ds(L, L)] = zero
                        return 0
                    lax.fori_loop(0, G, pzbody, 0)
                    pltpu.sync_copy(
                        batch_hbm.at[pl.ds(
                            lax.shift_right_logical(nodebase, 4),
                            BIN_SIZE // L)], batchbuf)

                    def pool_body(g, _):
                        bv = batchbuf[g, pl.ds(0, L)]
                        jb = g * L
                        for k in range(L):
                            gi = bv[k]
                            j = jb + k
                            p0 = pacc[gi, pl.ds(0, L)]
                            p1 = pacc[gi, pl.ds(L, L)]
                            pacc[gi, pl.ds(0, L)] = jnp.maximum(
                                p0, acc[j, pl.ds(0, L)])
                            pacc[gi, pl.ds(L, L)] = jnp.maximum(
                                p1, acc[j, pl.ds(L, L)])
                        return 0

                    lax.fori_loop(0, BIN_SIZE // L, pool_body, 0)

            if with_pool:
                @pl.when(jnp.logical_not(live))
                def _zero_partial():
                    def pzbody(i, _):
                        pacc[i, pl.ds(0, L)] = zero
                        pacc[i, pl.ds(L, L)] = zero
                        return 0
                    lax.fori_loop(0, G, pzbody, 0)

                pltpu.sync_copy(pacc, pool_hbm.at[pl.ds((r * NW + w) * G, G)])

    return scatmax


def _node_lin1(pos8, wu8, wv8, N_pad):
    BN = 2048

    def body(p_ref, wu_ref, wv_ref, u_ref, v_ref):
        p = p_ref[...]
        u_ref[...] = jnp.dot(p, wu_ref[...], preferred_element_type=jnp.float32, precision=lax.Precision.HIGHEST)
        v_ref[...] = jnp.dot(p, wv_ref[...], preferred_element_type=jnp.float32, precision=lax.Precision.HIGHEST)

    return pl.pallas_call(
        body,
        grid=(N_pad // BN,),
        in_specs=[pl.BlockSpec((BN, 8), lambda i: (i, 0)),
                  pl.BlockSpec((8, 32), lambda i: (0, 0)),
                  pl.BlockSpec((8, 32), lambda i: (0, 0))],
        out_specs=[pl.BlockSpec((BN, 32), lambda i: (i, 0)),
                   pl.BlockSpec((BN, 32), lambda i: (i, 0))],
        out_shape=[jax.ShapeDtypeStruct((N_pad, 32), jnp.float32),
                   jax.ShapeDtypeStruct((N_pad, 32), jnp.float32)],
    )(pos8, wu8, wv8)


def _node_lin2(h1, pos8, wtop, wbot8, N_pad):
    BN = 2048

    def body(h_ref, p_ref, wt_ref, wb_ref, u_ref, v_ref):
        t = jnp.dot(p_ref[...], wb_ref[...], preferred_element_type=jnp.float32, precision=lax.Precision.HIGHEST)
        u_ref[...] = jnp.dot(h_ref[...], wt_ref[...],
                             preferred_element_type=jnp.float32, precision=lax.Precision.HIGHEST) + t
        v_ref[...] = t

    return pl.pallas_call(
        body,
        grid=(N_pad // BN,),
        in_specs=[pl.BlockSpec((BN, 32), lambda i: (i, 0)),
                  pl.BlockSpec((BN, 8), lambda i: (i, 0)),
                  pl.BlockSpec((32, 32), lambda i: (0, 0)),
                  pl.BlockSpec((8, 32), lambda i: (0, 0))],
        out_specs=[pl.BlockSpec((BN, 32), lambda i: (i, 0)),
                   pl.BlockSpec((BN, 32), lambda i: (i, 0))],
        out_shape=[jax.ShapeDtypeStruct((N_pad, 32), jnp.float32),
                   jax.ShapeDtypeStruct((N_pad, 32), jnp.float32)],
    )(h1, pos8, wtop, wbot8)


def _edge_mlp(gu, gv, ba, wb, bb, E_pad):
    E4 = E_pad // 4
    BE = 384
    assert E4 % BE == 0 and BE % 8 == 0

    def body(gu_ref, gv_ref, ba_ref, wb_ref, bb_ref, m_ref):
        hid = jnp.maximum(gu_ref[...] - gv_ref[...] + ba_ref[...], 0.0)
        m_ref[...] = jnp.dot(hid, wb_ref[...],
                             preferred_element_type=jnp.float32, precision=lax.Precision.HIGHEST) + bb_ref[...]

    gu4 = gu.reshape(E4, 128)
    gv4 = gv.reshape(E4, 128)
    m4 = pl.pallas_call(
        body,
        grid=(E4 // BE,),
        in_specs=[pl.BlockSpec((BE, 128), lambda i: (i, 0)),
                  pl.BlockSpec((BE, 128), lambda i: (i, 0)),
                  pl.BlockSpec((1, 128), lambda i: (0, 0)),
                  pl.BlockSpec((128, 128), lambda i: (0, 0)),
                  pl.BlockSpec((1, 128), lambda i: (0, 0))],
        out_specs=pl.BlockSpec((BE, 128), lambda i: (i, 0)),
        out_shape=jax.ShapeDtypeStruct((E4, 128), jnp.float32),
    )(gu4, gv4, ba, wb, bb)
    return m4.reshape(E_pad, 32)


def _pool_final(partials, wc_pad, bc_pad, NPART, G=64):
    def body(p_ref, wc_ref, bc_ref, o_ref):
        p = p_ref[...].reshape(NPART, G, 32)
        pooled = jnp.max(p, axis=0)
        o_ref[...] = jnp.dot(pooled, wc_ref[...],
                             preferred_element_type=jnp.float32, precision=lax.Precision.HIGHEST) + bc_ref[...]

    return pl.pallas_call(
        body,
        out_shape=jax.ShapeDtypeStruct((G, 128), jnp.float32),
    )(partials, wc_pad, bc_pad)


def kernel(h, pos, batch, edge_index, W1a, b1a, W1b, b1b, W2a, b2a, W2b, b2b,
           Wc, bc):
    N = h.shape[0]
    E = edge_index.shape[1]
    G = 64
    assert E % (NW * 2000) == 0
    NBINS = (N + BIN_SIZE - 1) // BIN_SIZE
    N_pad = NBINS * BIN_SIZE
    E_pad = E + 2048
    ROUNDS = (NBINS + NW - 1) // NW

    src = edge_index[0]
    dst = edge_index[1]
    pos8 = jnp.zeros((N_pad, 8), jnp.float32).at[:N, :2].set(pos)
    batch_pad = jnp.zeros((N_pad,), jnp.int32).at[:N].set(batch)

    counts = _hist_kernel(E, NBINS)(dst)
    c3 = counts.reshape(NW, NBINS, L).transpose(1, 0, 2).reshape(-1)
    excl = jnp.concatenate([jnp.zeros((1,), jnp.int32),
                            jnp.cumsum(c3, dtype=jnp.int32)[:-1]])
    bases = excl.reshape(NBINS, NW, L).transpose(1, 0, 2).reshape(NW, NBINS * L)
    bin_tot = counts.reshape(NW, NBINS, L).sum(axis=(0, 2), dtype=jnp.int32)
    bin_start = jnp.concatenate([jnp.zeros((1,), jnp.int32),
                                 jnp.cumsum(bin_tot, dtype=jnp.int32)])
    bstart80 = jnp.full((80,), E, jnp.int32).at[:NBINS + 1].set(bin_start)
    bsrc, bdst = _binscatter_kernel(E, E_pad, NBINS)(src, dst, bases)
    bdst2 = bdst.reshape(E_pad // L, L)

    wu8 = jnp.zeros((8, 32), jnp.float32).at[:2].set(W1a[:2] + W1a[2:4])
    wv8 = jnp.zeros((8, 32), jnp.float32).at[:2].set(W1a[2:4])
    u1, v1 = _node_lin1(pos8, wu8, wv8, N_pad)
    eye4 = jnp.eye(4, dtype=jnp.float32)
    gu1, gv1 = _gather_kernel(E, E_pad, N_pad)(u1, v1, bsrc, bdst)
    m1 = _edge_mlp(gu1, gv1, jnp.tile(b1a, 4).reshape(1, 128),
                   jnp.kron(eye4, W1b), jnp.tile(b1b, 4).reshape(1, 128), E_pad)
    (h1,) = _scatmax_kernel(E, E_pad, N_pad, NBINS, with_pool=False)(
        m1, bdst2, bstart80)

    wbot8 = jnp.zeros((8, 32), jnp.float32).at[:2].set(W2a[32:34])
    u2, v2 = _node_lin2(h1, pos8, W2a[:32], wbot8, N_pad)
    gu2, gv2 = _gather_kernel(E, E_pad, N_pad)(u2, v2, bsrc, bdst)
    m2 = _edge_mlp(gu2, gv2, jnp.tile(b2a, 4).reshape(1, 128),
                   jnp.kron(eye4, W2b), jnp.tile(b2b, 4).reshape(1, 128), E_pad)
    _h2, partials = _scatmax_kernel(E, E_pad, N_pad, NBINS, with_pool=True)(
        m2, bdst2, bstart80, batch_pad.reshape(N_pad // L, L))

    wc_pad = jnp.zeros((32, 128), jnp.float32).at[:, :3].set(Wc)
    bc_pad = jnp.zeros((1, 128), jnp.float32).at[0, :3].set(bc)
    out = _pool_final(partials, wc_pad, bc_pad, NW * ROUNDS, G)
    return out[:, :3]


# speedup vs baseline: 3.3105x; 1.2444x over previous
import functools

import jax
import jax.numpy as jnp
from jax import lax
from jax.experimental import pallas as pl
from jax.experimental.pallas import tpu as pltpu
from jax.experimental.pallas import tpu_sc as plsc

NC = 2
NS = 16
NW = NC * NS
L = 16

BIN_SHIFT = 11
BIN_SIZE = 1 << BIN_SHIFT


def _mesh():
    return plsc.VectorSubcoreMesh(core_axis_name="c", subcore_axis_name="s")


def _wid():
    return lax.axis_index("s") * NC + lax.axis_index("c")


def _hist_kernel(E, NBINS):
    EPW = E // NW
    CH = 2000
    assert EPW % CH == 0 and CH % L == 0

    @functools.partial(
        pl.kernel, mesh=_mesh(),
        compiler_params=pltpu.CompilerParams(needs_layout_passes=False, use_tc_tiling_on_sc=False),
        out_type=jax.ShapeDtypeStruct((NW, NBINS * L), jnp.int32),
        scratch_types=[
            pltpu.VMEM((CH,), jnp.int32),
            pltpu.VMEM((NBINS * L,), jnp.int32),
        ],
    )
    def hist(dst_hbm, counts_hbm, dstbuf, cnt):
        w = _wid()
        lane = lax.iota(jnp.int32, L)
        zero = jnp.zeros((L,), jnp.int32)
        one = jnp.ones((L,), jnp.int32)
        for i in range(NBINS):
            cnt[pl.ds(i * L, L)] = zero
        base = w * EPW

        def chunk_body(c, _):
            pltpu.sync_copy(dst_hbm.at[pl.ds(base + c * CH, CH)], dstbuf)
            for g in range(CH // L):
                d = dstbuf[pl.ds(g * L, L)]
                b = lax.shift_right_logical(d, BIN_SHIFT)
                idxv = b * L + lane
                plsc.addupdate_scatter(cnt, [idxv], one)
            return 0

        lax.fori_loop(0, EPW // CH, chunk_body, 0)
        pltpu.sync_copy(cnt, counts_hbm.at[w])

    return hist


def _binscatter_kernel(E, E_pad, NBINS):
    EPW = E // NW
    CH = 2000
    assert EPW % CH == 0

    @functools.partial(
        pl.kernel, mesh=_mesh(),
        compiler_params=pltpu.CompilerParams(needs_layout_passes=False, use_tc_tiling_on_sc=False),
        out_type=[jax.ShapeDtypeStruct((E_pad,), jnp.int32),
                  jax.ShapeDtypeStruct((E_pad,), jnp.int32)],
        scratch_types=[
            pltpu.VMEM((CH,), jnp.int32),
            pltpu.VMEM((CH,), jnp.int32),
            pltpu.VMEM((NBINS * L,), jnp.int32),
            pltpu.VMEM((NBINS * L,), jnp.int32),
            pltpu.SemaphoreType.DMA,
        ],
    )
    def binscatter(dst_hbm, base_hbm, bdst_hbm, pos_hbm,
                   dstbuf, posbuf, run, basebuf, sem1):
        w = _wid()
        lane = lax.iota(jnp.int32, L)
        zero = jnp.zeros((L,), jnp.int32)
        one = jnp.ones((L,), jnp.int32)
        pltpu.sync_copy(base_hbm.at[w], basebuf)
        for i in range(NBINS):
            run[pl.ds(i * L, L)] = zero
        base = w * EPW

        def chunk_body(c, _):
            pltpu.sync_copy(dst_hbm.at[pl.ds(base + c * CH, CH)], dstbuf)
            for g in range(CH // L):
                d = dstbuf[pl.ds(g * L, L)]
                b = lax.shift_right_logical(d, BIN_SHIFT)
                idxv = b * L + lane
                bs = plsc.load_gather(basebuf, [idxv])
                cr = plsc.load_gather(run, [idxv])
                plsc.store_scatter(run, [idxv], cr + one)
                posbuf[pl.ds(g * L, L)] = bs + cr
            c2 = pltpu.async_copy(dstbuf, bdst_hbm.at[posbuf], sem1)
            pltpu.sync_copy(posbuf, pos_hbm.at[pl.ds(base + c * CH, CH)])
            c2.wait()
            return 0

        lax.fori_loop(0, EPW // CH, chunk_body, 0)

    return binscatter


def _gather_kernel(E, E_pad, N_pad):
    EPW = E // NW
    CH = 1000
    assert EPW % CH == 0

    @functools.partial(
        pl.kernel, mesh=_mesh(),
        compiler_params=pltpu.CompilerParams(needs_layout_passes=False, use_tc_tiling_on_sc=False),
        out_type=[jax.ShapeDtypeStruct((E_pad, 32), jnp.float32),
                  jax.ShapeDtypeStruct((E_pad, 32), jnp.float32)],
        scratch_types=[
            pltpu.VMEM((CH,), jnp.int32),
            pltpu.VMEM((CH,), jnp.int32),
            pltpu.VMEM((CH,), jnp.int32),
            pltpu.VMEM((CH, 32), jnp.float32),
            pltpu.VMEM((CH, 32), jnp.float32),
            pltpu.SemaphoreType.DMA,
            pltpu.SemaphoreType.DMA,
            pltpu.SemaphoreType.DMA,
            pltpu.SemaphoreType.DMA,
        ],
    )
    def gather(u_hbm, v_hbm, src_hbm, dst_hbm, pos_hbm, gu_hbm, gv_hbm,
               idxs, idxd, idxp, gubuf, gvbuf, sem1, sem2, sem3, sem4):
        w = _wid()
        base = w * EPW

        def chunk_body(c, _):
            off = base + c * CH
            pltpu.sync_copy(src_hbm.at[pl.ds(off, CH)], idxs)
            pltpu.sync_copy(dst_hbm.at[pl.ds(off, CH)], idxd)
            pltpu.sync_copy(pos_hbm.at[pl.ds(off, CH)], idxp)
            c1 = pltpu.async_copy(u_hbm.at[idxs], gubuf, sem1)
            c2 = pltpu.async_copy(v_hbm.at[idxd], gvbuf, sem2)
            c1.wait()
            c2.wait()
            c3 = pltpu.async_copy(gubuf, gu_hbm.at[idxp], sem3)
            c4 = pltpu.async_copy(gvbuf, gv_hbm.at[idxp], sem4)
            c3.wait()
            c4.wait()
            return 0

        lax.fori_loop(0, EPW // CH, chunk_body, 0)

    return gather


def _scatmax_kernel(E, E_pad, N_pad, NBINS, with_pool, G=64):
    CH = 512
    ROUNDS = (NBINS + NW - 1) // NW

    out_type = [jax.ShapeDtypeStruct((N_pad, 32), jnp.float32)]
    if with_pool:
        out_type.append(jax.ShapeDtypeStruct((NW * ROUNDS * G, 32), jnp.float32))
    scratch = [
        pltpu.VMEM((BIN_SIZE, 32), jnp.float32),
        pltpu.VMEM((CH // L, L), jnp.int32),
        pltpu.VMEM((CH, 32), jnp.float32),
        pltpu.VMEM((80,), jnp.int32),
    ]
    if with_pool:
        scratch.append(pltpu.VMEM((G, 32), jnp.float32))
        scratch.append(pltpu.VMEM((BIN_SIZE // L, L), jnp.int32))

    @functools.partial(pl.kernel, mesh=_mesh(), out_type=out_type,
                       compiler_params=pltpu.CompilerParams(needs_layout_passes=False, use_tc_tiling_on_sc=False),
                       scratch_types=scratch)
    def scatmax(*refs):
        if with_pool:
            (m_hbm, bdst_hbm, bstart_hbm, batch_hbm, h_hbm, pool_hbm,
             acc, dstbuf, mbuf, bsbuf, pacc, batchbuf) = refs
        else:
            (m_hbm, bdst_hbm, bstart_hbm, h_hbm,
             acc, dstbuf, mbuf, bsbuf) = refs
        w = _wid()
        zero = jnp.zeros((L,), jnp.float32)
        pltpu.sync_copy(bstart_hbm, bsbuf)
        for r in range(ROUNDS):
            bin_id = w + NW * r
            live = bin_id < NBINS

            @pl.when(live)
            def _process():
                def zbody(i, _):
                    for k in range(4):
                        acc[i * 2 + (k // 2), pl.ds((k % 2) * L, L)] = zero
                    return 0
                lax.fori_loop(0, BIN_SIZE // 2, zbody, 0)

                ebounds = bsbuf[pl.ds(bin_id, 16)]
                e0 = ebounds[0]
                e1 = ebounds[1]
                abase = e0 - jnp.bitwise_and(e0, L - 1)
                nch = (e1 - abase + CH - 1) // CH
                nodebase = bin_id * BIN_SIZE

                def chunk_body(c, _):
                    cstart = abase + c * CH
                    pltpu.sync_copy(
                        bdst_hbm.at[pl.ds(lax.shift_right_logical(cstart, 4),
                                          CH // L)], dstbuf)
                    pltpu.sync_copy(m_hbm.at[pl.ds(cstart, CH)], mbuf)
                    lo = jnp.maximum(e0, cstart) - cstart
                    hi = jnp.minimum(e1, cstart + CH) - cstart
                    glo = lax.shift_right_logical(lo, 4)
                    ghi = lax.shift_right_logical(hi + L - 1, 4)

                    def group_body(g, _):
                        dv = dstbuf[g, pl.ds(0, L)] - nodebase
                        jb = g * L
                        for k in range(L):
                            j = jb + k

                            @pl.when(jnp.logical_and(j >= lo, j < hi))
                            def _rmw():
                                dl = dv[k]
                                a0 = acc[dl, pl.ds(0, L)]
                                a1 = acc[dl, pl.ds(L, L)]
                                m0 = mbuf[j, pl.ds(0, L)]
                                m1 = mbuf[j, pl.ds(L, L)]
                                acc[dl, pl.ds(0, L)] = jnp.maximum(a0, m0)
                                acc[dl, pl.ds(L, L)] = jnp.maximum(a1, m1)
                        return 0

                    lax.fori_loop(glo, ghi, group_body, 0)
                    return 0

                lax.fori_loop(0, nch, chunk_body, 0)
                pltpu.sync_copy(acc, h_hbm.at[pl.ds(nodebase, BIN_SIZE)])

                if with_pool:
                    def pzbody(i, _):
                        pacc[i, pl.ds(0, L)] = zero
                        pacc[i, pl.ds(L, L)] = zero
                        return 0
                    lax.fori_loop(0, G, pzbody, 0)
                    pltpu.sync_copy(
                        batch_hbm.at[pl.ds(
                            lax.shift_right_logical(nodebase, 4),
                            BIN_SIZE // L)], batchbuf)

                    def pool_body(g, _):
                        bv = batchbuf[g, pl.ds(0, L)]
                        jb = g * L
                        for k in range(L):
                            gi = bv[k]
                            j = jb + k
                            p0 = pacc[gi, pl.ds(0, L)]
                            p1 = pacc[gi, pl.ds(L, L)]
                            pacc[gi, pl.ds(0, L)] = jnp.maximum(
                                p0, acc[j, pl.ds(0, L)])
                            pacc[gi, pl.ds(L, L)] = jnp.maximum(
                                p1, acc[j, pl.ds(L, L)])
                        return 0

                    lax.fori_loop(0, BIN_SIZE // L, pool_body, 0)

            if with_pool:
                @pl.when(jnp.logical_not(live))
                def _zero_partial():
                    def pzbody(i, _):
                        pacc[i, pl.ds(0, L)] = zero
                        pacc[i, pl.ds(L, L)] = zero
                        return 0
                    lax.fori_loop(0, G, pzbody, 0)

                pltpu.sync_copy(pacc, pool_hbm.at[pl.ds((r * NW + w) * G, G)])

    return scatmax


def _node_lin1(pos8, wu8, wv8, N_pad):
    BN = 2048

    def body(p_ref, wu_ref, wv_ref, u_ref, v_ref):
        p = p_ref[...]
        u_ref[...] = jnp.dot(p, wu_ref[...], preferred_element_type=jnp.float32, precision=lax.Precision.HIGHEST)
        v_ref[...] = jnp.dot(p, wv_ref[...], preferred_element_type=jnp.float32, precision=lax.Precision.HIGHEST)

    return pl.pallas_call(
        body,
        grid=(N_pad // BN,),
        in_specs=[pl.BlockSpec((BN, 8), lambda i: (i, 0)),
                  pl.BlockSpec((8, 32), lambda i: (0, 0)),
                  pl.BlockSpec((8, 32), lambda i: (0, 0))],
        out_specs=[pl.BlockSpec((BN, 32), lambda i: (i, 0)),
                   pl.BlockSpec((BN, 32), lambda i: (i, 0))],
        out_shape=[jax.ShapeDtypeStruct((N_pad, 32), jnp.float32),
                   jax.ShapeDtypeStruct((N_pad, 32), jnp.float32)],
    )(pos8, wu8, wv8)


def _node_lin2(h1, pos8, wtop, wbot8, N_pad):
    BN = 2048

    def body(h_ref, p_ref, wt_ref, wb_ref, u_ref, v_ref):
        t = jnp.dot(p_ref[...], wb_ref[...], preferred_element_type=jnp.float32, precision=lax.Precision.HIGHEST)
        u_ref[...] = jnp.dot(h_ref[...], wt_ref[...],
                             preferred_element_type=jnp.float32, precision=lax.Precision.HIGHEST) + t
        v_ref[...] = t

    return pl.pallas_call(
        body,
        grid=(N_pad // BN,),
        in_specs=[pl.BlockSpec((BN, 32), lambda i: (i, 0)),
                  pl.BlockSpec((BN, 8), lambda i: (i, 0)),
                  pl.BlockSpec((32, 32), lambda i: (0, 0)),
                  pl.BlockSpec((8, 32), lambda i: (0, 0))],
        out_specs=[pl.BlockSpec((BN, 32), lambda i: (i, 0)),
                   pl.BlockSpec((BN, 32), lambda i: (i, 0))],
        out_shape=[jax.ShapeDtypeStruct((N_pad, 32), jnp.float32),
                   jax.ShapeDtypeStruct((N_pad, 32), jnp.float32)],
    )(h1, pos8, wtop, wbot8)


def _edge_mlp(gu, gv, ba, wb, bb, E_pad):
    E4 = E_pad // 4
    BE = 384
    assert E4 % BE == 0 and BE % 8 == 0

    def body(gu_ref, gv_ref, ba_ref, wb_ref, bb_ref, m_ref):
        hid = jnp.maximum(gu_ref[...] - gv_ref[...] + ba_ref[...], 0.0)
        m_ref[...] = jnp.dot(hid, wb_ref[...],
                             preferred_element_type=jnp.float32, precision=lax.Precision.HIGHEST) + bb_ref[...]

    gu4 = gu.reshape(E4, 128)
    gv4 = gv.reshape(E4, 128)
    m4 = pl.pallas_call(
        body,
        grid=(E4 // BE,),
        in_specs=[pl.BlockSpec((BE, 128), lambda i: (i, 0)),
                  pl.BlockSpec((BE, 128), lambda i: (i, 0)),
                  pl.BlockSpec((1, 128), lambda i: (0, 0)),
                  pl.BlockSpec((128, 128), lambda i: (0, 0)),
                  pl.BlockSpec((1, 128), lambda i: (0, 0))],
        out_specs=pl.BlockSpec((BE, 128), lambda i: (i, 0)),
        out_shape=jax.ShapeDtypeStruct((E4, 128), jnp.float32),
    )(gu4, gv4, ba, wb, bb)
    return m4.reshape(E_pad, 32)


def _pool_final(partials, wc_pad, bc_pad, NPART, G=64):
    def body(p_ref, wc_ref, bc_ref, o_ref):
        p = p_ref[...].reshape(NPART, G, 32)
        pooled = jnp.max(p, axis=0)
        o_ref[...] = jnp.dot(pooled, wc_ref[...],
                             preferred_element_type=jnp.float32, precision=lax.Precision.HIGHEST) + bc_ref[...]

    return pl.pallas_call(
        body,
        out_shape=jax.ShapeDtypeStruct((G, 128), jnp.float32),
    )(partials, wc_pad, bc_pad)


def kernel(h, pos, batch, edge_index, W1a, b1a, W1b, b1b, W2a, b2a, W2b, b2b,
           Wc, bc):
    N = h.shape[0]
    E = edge_index.shape[1]
    G = 64
    assert E % (NW * 2000) == 0
    NBINS = (N + BIN_SIZE - 1) // BIN_SIZE
    N_pad = NBINS * BIN_SIZE
    E_pad = E + 2048
    ROUNDS = (NBINS + NW - 1) // NW

    src = edge_index[0]
    dst = edge_index[1]
    pos8 = jnp.zeros((N_pad, 8), jnp.float32).at[:N, :2].set(pos)
    batch_pad = jnp.zeros((N_pad,), jnp.int32).at[:N].set(batch)

    counts = _hist_kernel(E, NBINS)(dst)
    c3 = counts.reshape(NW, NBINS, L).transpose(1, 0, 2).reshape(-1)
    excl = jnp.concatenate([jnp.zeros((1,), jnp.int32),
                            jnp.cumsum(c3, dtype=jnp.int32)[:-1]])
    bases = excl.reshape(NBINS, NW, L).transpose(1, 0, 2).reshape(NW, NBINS * L)
    bin_tot = counts.reshape(NW, NBINS, L).sum(axis=(0, 2), dtype=jnp.int32)
    bin_start = jnp.concatenate([jnp.zeros((1,), jnp.int32),
                                 jnp.cumsum(bin_tot, dtype=jnp.int32)])
    bstart80 = jnp.full((80,), E, jnp.int32).at[:NBINS + 1].set(bin_start)
    bdst, eposn = _binscatter_kernel(E, E_pad, NBINS)(dst, bases)
    bdst2 = bdst.reshape(E_pad // L, L)

    wu8 = jnp.zeros((8, 32), jnp.float32).at[:2].set(W1a[:2] + W1a[2:4])
    wv8 = jnp.zeros((8, 32), jnp.float32).at[:2].set(W1a[2:4])
    u1, v1 = _node_lin1(pos8, wu8, wv8, N_pad)
    eye4 = jnp.eye(4, dtype=jnp.float32)
    gu1, gv1 = _gather_kernel(E, E_pad, N_pad)(u1, v1, src, dst, eposn)
    m1 = _edge_mlp(gu1, gv1, jnp.tile(b1a, 4).reshape(1, 128),
                   jnp.kron(eye4, W1b), jnp.tile(b1b, 4).reshape(1, 128), E_pad)
    (h1,) = _scatmax_kernel(E, E_pad, N_pad, NBINS, with_pool=False)(
        m1, bdst2, bstart80)

    wbot8 = jnp.zeros((8, 32), jnp.float32).at[:2].set(W2a[32:34])
    u2, v2 = _node_lin2(h1, pos8, W2a[:32], wbot8, N_pad)
    gu2, gv2 = _gather_kernel(E, E_pad, N_pad)(u2, v2, src, dst, eposn)
    m2 = _edge_mlp(gu2, gv2, jnp.tile(b2a, 4).reshape(1, 128),
                   jnp.kron(eye4, W2b), jnp.tile(b2b, 4).reshape(1, 128), E_pad)
    _h2, partials = _scatmax_kernel(E, E_pad, N_pad, NBINS, with_pool=True)(
        m2, bdst2, bstart80, batch_pad.reshape(N_pad // L, L))

    wc_pad = jnp.zeros((32, 128), jnp.float32).at[:, :3].set(Wc)
    bc_pad = jnp.zeros((1, 128), jnp.float32).at[0, :3].set(bc)
    out = _pool_final(partials, wc_pad, bc_pad, NW * ROUNDS, G)
    return out[:, :3]
